# Initial kernel scaffold; baseline (speedup 1.0000x reference)
import jax
import jax.numpy as jnp
from jax.experimental import pallas as pl


def kernel(user_emb, item_emb, edge_weight, users, items, edge_index):
    raise NotImplementedError("write your pallas kernel here")


# speedup vs baseline: 4.7880x; 4.7880x over previous
import jax
import jax.numpy as jnp
from jax import lax
from jax.experimental import pallas as pl
from jax.experimental.pallas import tpu as pltpu
from jax.experimental.pallas import tpu_sc as plsc

N_USERS = 5000
N_ITEMS = 5000
NN = N_USERS + N_ITEMS
NE = 320000
D = 128
NG = D // 16
N_LAYERS = 3
NC, NS, LANES = 2, 16, 16
NW = NC * NS
ESL = NE // NS
NPASS = 2
PESL = ESL // NPASS
HALF = NN // NC
G = 128
ACC_ROWS = HALF + 16
SEL_PAD = PESL + G + 16
RPT = 312
WBC = 24
EBLK = 2000

_f32 = jnp.float32
_i32 = jnp.int32


def _layer_body(x_hbm, dst_hbm, src_hbm, w_hbm, y_hbm, xo_hbm, yo_hbm,
                dst_v, src_v, w_v, sdst, ssrc, sw,
                gidx0, gidx1, sidx0, sidx1, rows0, rows1,
                zbuf, abuf, ybuf, acc_sh, gsem0, gsem1):
    c = lax.axis_index("c")
    s = lax.axis_index("s")
    ebase = s * ESL

    zero_f = jnp.zeros((16,), _f32)
    zero_i = jnp.zeros((16,), _i32)
    trash = jnp.full((16,), HALF, _i32)
    lane = lax.iota(_i32, 16)

    for r in range(8):
        for j in range(NG):
            zbuf[r, pl.ds(j * 16, 16)] = zero_f
    row0 = s * RPT

    def _zc(i, _):
        pltpu.sync_copy(zbuf, acc_sh.at[pl.ds(row0 + i * 8, 8)])
        return 0

    n_z = jnp.where(s == NS - 1, RPT // 8 + 1, RPT // 8)
    lax.fori_loop(0, n_z, _zc, 0)
    plsc.subcore_barrier()

    halfbase = c * HALF
    dump = jnp.full((16,), SEL_PAD - 16, _i32) + lane

    gidx = (gidx0, gidx1)
    sidx = (sidx0, sidx1)
    rows = (rows0, rows1)
    gsem = (gsem0, gsem1)

    def _prep(k, b):
        base = k * G

        def _ic(i, _):
            gidx[b][pl.ds(i * 16, 16)] = ssrc[pl.ds(base + i * 16, 16)]
            sidx[b][pl.ds(i * 16, 16)] = sdst[pl.ds(base + i * 16, 16)]
            return 0

        lax.fori_loop(0, G // 16, _ic, 0)
        pltpu.async_copy(x_hbm.at[gidx[b]], rows[b], gsem[b])

    def _scale(k, b):
        def _eb(i, _):
            wv = sw[pl.ds(k * G + i * 16, 16)]
            for l in range(16):
                wl = jnp.full((16,), wv[l])
                for j in range(NG):
                    rows[b][i * 16 + l, pl.ds(j * 16, 16)] = (
                        rows[b][i * 16 + l, pl.ds(j * 16, 16)] * wl)
            return 0

        lax.fori_loop(0, G // 16, _eb, 0)

    for p in range(NPASS):
        pbase = ebase + p * PESL

        def _blk(bi, cnt):
            pltpu.sync_copy(dst_hbm.at[pl.ds(pbase + bi * EBLK, EBLK)], dst_v)
            pltpu.sync_copy(src_hbm.at[pl.ds(pbase + bi * EBLK, EBLK)], src_v)
            pltpu.sync_copy(w_hbm.at[pl.ds(pbase + bi * EBLK, EBLK)], w_v)

            def _fb(i, cnt):
                dv = dst_v[pl.ds(i * 16, 16)]
                loc = dv - halfbase
                m = (loc >= 0) & (loc < HALF)
                cum = plsc.cumsum(m.astype(_i32))
                pos = jnp.where(m, cnt + cum - 1, dump)
                plsc.store_scatter(sdst, [pos], loc)
                plsc.store_scatter(ssrc, [pos], src_v[pl.ds(i * 16, 16)])
                plsc.store_scatter(sw, [pos], w_v[pl.ds(i * 16, 16)])
                return cnt + cum[15]

            return lax.fori_loop(0, EBLK // 16, _fb, cnt)

        cnt = lax.fori_loop(0, PESL // EBLK, _blk, _i32(0))

        for i in range(G // 16):
            pidx = cnt + i * 16 + lane
            plsc.store_scatter(sdst, [pidx], trash)
            plsc.store_scatter(ssrc, [pidx], zero_i)
            plsc.store_scatter(sw, [pidx], zero_f)

        nch = jnp.maximum((cnt + G - 1) // G, 1)

        _prep(_i32(0), 0)

        def _round(r, _):
            for b in range(2):
                k = r * 2 + b

                @pl.when(k < nch)
                def _():
                    pltpu.make_async_copy(
                        x_hbm.at[gidx[b]], rows[b], gsem[b]).wait()

                    @pl.when(k + 1 < nch)
                    def _():
                        _prep(k + 1, 1 - b)

                    _scale(k, b)
                    pltpu.sync_copy(rows[b], acc_sh.at[sidx[b]], add=True)
            return 0

        lax.fori_loop(0, (nch + 1) // 2, _round, 0)

    plsc.subcore_barrier()

    grow0 = c * HALF + row0

    def _wb(i, _):
        r0 = row0 + i * WBC
        g0 = grow0 + i * WBC
        pltpu.sync_copy(acc_sh.at[pl.ds(r0, WBC)], abuf)
        pltpu.sync_copy(y_hbm.at[pl.ds(g0, WBC)], ybuf)

        def _ar(r, _):
            for j in range(NG):
                ybuf[r, pl.ds(j * 16, 16)] = (
                    ybuf[r, pl.ds(j * 16, 16)] + abuf[r, pl.ds(j * 16, 16)])
            return 0

        lax.fori_loop(0, WBC, _ar, 0)
        pltpu.sync_copy(abuf, xo_hbm.at[pl.ds(g0, WBC)])
        pltpu.sync_copy(ybuf, yo_hbm.at[pl.ds(g0, WBC)])
        return 0

    lax.fori_loop(0, RPT // WBC, _wb, 0)

    @pl.when(s == NS - 1)
    def _tail():
        r0 = RPT * NS
        g0 = c * HALF + r0
        pltpu.sync_copy(acc_sh.at[pl.ds(r0, 8)], abuf.at[pl.ds(0, 8)])
        pltpu.sync_copy(y_hbm.at[pl.ds(g0, 8)], ybuf.at[pl.ds(0, 8)])

        def _ar(r, _):
            for j in range(NG):
                ybuf[r, pl.ds(j * 16, 16)] = (
                    ybuf[r, pl.ds(j * 16, 16)] + abuf[r, pl.ds(j * 16, 16)])
            return 0

        lax.fori_loop(0, 8, _ar, 0)
        pltpu.sync_copy(abuf.at[pl.ds(0, 8)], xo_hbm.at[pl.ds(g0, 8)])
        pltpu.sync_copy(ybuf.at[pl.ds(0, 8)], yo_hbm.at[pl.ds(g0, 8)])


def _gamma_body(y_hbm, u_hbm, i_hbm, out_hbm,
                uidx, iidx, urows, irows, obuf, n_pairs):
    c = lax.axis_index("c")
    s = lax.axis_index("s")
    wid = s * NC + c
    ppt = n_pairs // NW
    pbase = wid * ppt
    lane = lax.iota(_i32, 16)
    nusers = jnp.full((16,), N_USERS, _i32)

    for cc in range(ppt // G):
        base = pbase + cc * G
        pltpu.sync_copy(u_hbm.at[pl.ds(base, G)], uidx)
        pltpu.sync_copy(i_hbm.at[pl.ds(base, G)], iidx)

        def _ib(i, _):
            iidx[pl.ds(i * 16, 16)] = iidx[pl.ds(i * 16, 16)] + nusers
            return 0

        lax.fori_loop(0, G // 16, _ib, 0)
        pltpu.sync_copy(y_hbm.at[uidx], urows)
        pltpu.sync_copy(y_hbm.at[iidx], irows)

        def _gb(g, _):
            out = jnp.zeros((16,), _f32)
            for l in range(16):
                e = g * 16 + l
                acc = jnp.zeros((16,), _f32)
                for j in range(NG):
                    acc = acc + (urows[e, pl.ds(j * 16, 16)]
                                 * irows[e, pl.ds(j * 16, 16)])
                ssum = jnp.sum(acc)
                out = jnp.where(lane == l, jnp.full((16,), ssum), out)
            obuf[pl.ds(cc * G + g * 16, 16)] = out * (1.0 / 64.0)
            return 0

        lax.fori_loop(0, G // 16, _gb, 0)

    pltpu.sync_copy(obuf, out_hbm.at[pl.ds(pbase, ppt)])


def _make_layer():
    mesh = plsc.VectorSubcoreMesh(core_axis_name="c", subcore_axis_name="s")
    return pl.kernel(
        _layer_body,
        out_type=(jax.ShapeDtypeStruct((NN, D), _f32),
                  jax.ShapeDtypeStruct((NN, D), _f32)),
        mesh=mesh,
        compiler_params=pltpu.CompilerParams(needs_layout_passes=False),
        scratch_types=[
            pltpu.VMEM((EBLK,), _i32),
            pltpu.VMEM((EBLK,), _i32),
            pltpu.VMEM((EBLK,), _f32),
            pltpu.VMEM((SEL_PAD,), _i32),
            pltpu.VMEM((SEL_PAD,), _i32),
            pltpu.VMEM((SEL_PAD,), _f32),
            pltpu.VMEM((G,), _i32),
            pltpu.VMEM((G,), _i32),
            pltpu.VMEM((G,), _i32),
            pltpu.VMEM((G,), _i32),
            pltpu.VMEM((G, D), _f32),
            pltpu.VMEM((G, D), _f32),
            pltpu.VMEM((8, D), _f32),
            pltpu.VMEM((WBC, D), _f32),
            pltpu.VMEM((WBC, D), _f32),
            pltpu.VMEM_SHARED((ACC_ROWS, D), _f32),
            pltpu.SemaphoreType.DMA,
            pltpu.SemaphoreType.DMA,
        ],
    )


def _make_gamma(n_pairs):
    mesh = plsc.VectorSubcoreMesh(core_axis_name="c", subcore_axis_name="s")
    body = lambda *refs: _gamma_body(*refs, n_pairs=n_pairs)
    return pl.kernel(
        body,
        out_type=jax.ShapeDtypeStruct((n_pairs,), _f32),
        mesh=mesh,
        compiler_params=pltpu.CompilerParams(needs_layout_passes=False),
        scratch_types=[
            pltpu.VMEM((G,), _i32),
            pltpu.VMEM((G,), _i32),
            pltpu.VMEM((G, D), _f32),
            pltpu.VMEM((G, D), _f32),
            pltpu.VMEM((n_pairs // NW,), _f32),
        ],
    )


def kernel(user_emb, item_emb, edge_weight, users, items, edge_index):
    x0 = jnp.concatenate([user_emb, item_emb], axis=0)
    dst = edge_index[0].astype(_i32)
    src = edge_index[1].astype(_i32)
    w = edge_weight
    layer = _make_layer()
    x, y = x0, x0
    for _ in range(N_LAYERS):
        x, y = layer(x, dst, src, w, y)
    gamma = _make_gamma(users.shape[0])
    return gamma(y, users.astype(_i32), items.astype(_i32))


# speedup vs baseline: 4.9586x; 1.0356x over previous
import jax
import jax.numpy as jnp
from jax import lax
from jax.experimental import pallas as pl
from jax.experimental.pallas import tpu as pltpu
from jax.experimental.pallas import tpu_sc as plsc

N_USERS = 5000
N_ITEMS = 5000
NN = N_USERS + N_ITEMS
NE = 320000
D = 128
NG = D // 16
N_LAYERS = 3
NC, NS, LANES = 2, 16, 16
NW = NC * NS
ESL = NE // NS
NPASS = 2
PESL = ESL // NPASS
HALF = NN // NC
G = 128
ACC_ROWS = HALF + 16
SEL_PAD = PESL + G + 16
RPT = 312
ZC = 24
EBLK = 2000
GP = 64

_f32 = jnp.float32
_i32 = jnp.int32


def _layer_body(x_hbm, dst_hbm, src_hbm, w_hbm, xo_hbm,
                dst_v, src_v, w_v, sdst, ssrc, sw,
                gidx0, gidx1, sidx0, sidx1, rows0, rows1,
                zbuf, acc_sh, gsem0, gsem1, ssem0, ssem1, zsem):
    c = lax.axis_index("c")
    s = lax.axis_index("s")
    ebase = s * ESL

    zero_f = jnp.zeros((16,), _f32)
    zero_i = jnp.zeros((16,), _i32)
    trash = jnp.full((16,), HALF, _i32)
    lane = lax.iota(_i32, 16)

    for r in range(ZC):
        for j in range(NG):
            zbuf[r, pl.ds(j * 16, 16)] = zero_f
    row0 = s * RPT

    def _zc(i, _):
        pltpu.async_copy(zbuf, acc_sh.at[pl.ds(row0 + i * ZC, ZC)], zsem)
        return 0

    lax.fori_loop(0, RPT // ZC, _zc, 0)

    @pl.when(s == NS - 1)
    def _ztail():
        pltpu.async_copy(zbuf.at[pl.ds(0, 8)],
                         acc_sh.at[pl.ds(RPT * NS, 8)], zsem)

    def _zw(i, _):
        pltpu.make_async_copy(zbuf, acc_sh.at[pl.ds(row0, ZC)], zsem).wait()
        return 0

    lax.fori_loop(0, RPT // ZC, _zw, 0)

    @pl.when(s == NS - 1)
    def _ztailw():
        pltpu.make_async_copy(zbuf.at[pl.ds(0, 8)],
                              acc_sh.at[pl.ds(RPT * NS, 8)], zsem).wait()

    plsc.subcore_barrier()

    halfbase = c * HALF
    dump = jnp.full((16,), SEL_PAD - 16, _i32) + lane

    gidx = (gidx0, gidx1)
    sidx = (sidx0, sidx1)
    rows = (rows0, rows1)
    gsem = (gsem0, gsem1)
    ssem = (ssem0, ssem1)

    def _prep(k, b):
        base = k * G

        def _ic(i, _):
            gidx[b][pl.ds(i * 16, 16)] = ssrc[pl.ds(base + i * 16, 16)]
            sidx[b][pl.ds(i * 16, 16)] = sdst[pl.ds(base + i * 16, 16)]
            return 0

        lax.fori_loop(0, G // 16, _ic, 0)
        pltpu.async_copy(x_hbm.at[gidx[b]], rows[b], gsem[b])

    def _scale(k, b):
        def _eb(i, _):
            wv = sw[pl.ds(k * G + i * 16, 16)]
            for l in range(16):
                wl = jnp.full((16,), wv[l])
                for j in range(NG):
                    rows[b][i * 16 + l, pl.ds(j * 16, 16)] = (
                        rows[b][i * 16 + l, pl.ds(j * 16, 16)] * wl)
            return 0

        lax.fori_loop(0, G // 16, _eb, 0)

    for p in range(NPASS):
        pbase = ebase + p * PESL

        def _blk(bi, cnt):
            pltpu.async_copy(dst_hbm.at[pl.ds(pbase + bi * EBLK, EBLK)],
                             dst_v, zsem)
            pltpu.async_copy(src_hbm.at[pl.ds(pbase + bi * EBLK, EBLK)],
                             src_v, zsem)
            pltpu.async_copy(w_hbm.at[pl.ds(pbase + bi * EBLK, EBLK)],
                             w_v, zsem)
            pltpu.make_async_copy(dst_hbm.at[pl.ds(pbase, EBLK)],
                                  dst_v, zsem).wait()
            pltpu.make_async_copy(src_hbm.at[pl.ds(pbase, EBLK)],
                                  src_v, zsem).wait()
            pltpu.make_async_copy(w_hbm.at[pl.ds(pbase, EBLK)],
                                  w_v, zsem).wait()

            def _fb(i, cnt):
                dv = dst_v[pl.ds(i * 16, 16)]
                loc = dv - halfbase
                m = (loc >= 0) & (loc < HALF)
                cum = plsc.cumsum(m.astype(_i32))
                pos = jnp.where(m, cnt + cum - 1, dump)
                plsc.store_scatter(sdst, [pos], loc)
                plsc.store_scatter(ssrc, [pos], src_v[pl.ds(i * 16, 16)])
                plsc.store_scatter(sw, [pos], w_v[pl.ds(i * 16, 16)])
                return cnt + cum[15]

            return lax.fori_loop(0, EBLK // 16, _fb, cnt)

        cnt = lax.fori_loop(0, PESL // EBLK, _blk, _i32(0))

        for i in range(G // 16):
            pidx = cnt + i * 16 + lane
            plsc.store_scatter(sdst, [pidx], trash)
            plsc.store_scatter(ssrc, [pidx], zero_i)
            plsc.store_scatter(sw, [pidx], zero_f)

        nch = jnp.maximum((cnt + G - 1) // G, 1)

        _prep(_i32(0), 0)

        def _round(r, _):
            for b in range(2):
                k = r * 2 + b

                @pl.when(k < nch)
                def _():
                    pltpu.make_async_copy(
                        x_hbm.at[gidx[b]], rows[b], gsem[b]).wait()

                    @pl.when(k > 0)
                    def _():
                        pltpu.make_async_copy(
                            rows[1 - b], acc_sh.at[sidx[1 - b]],
                            ssem[1 - b]).wait()

                    @pl.when(k + 1 < nch)
                    def _():
                        _prep(k + 1, 1 - b)

                    _scale(k, b)
                    pltpu.async_copy(rows[b], acc_sh.at[sidx[b]],
                                     ssem[b], add=True)
            return 0

        lax.fori_loop(0, (nch + 1) // 2, _round, 0)

        @pl.when((nch - 1) % 2 == 0)
        def _d0():
            pltpu.make_async_copy(rows0, acc_sh.at[sidx0], ssem0).wait()

        @pl.when((nch - 1) % 2 == 1)
        def _d1():
            pltpu.make_async_copy(rows1, acc_sh.at[sidx1], ssem1).wait()

    plsc.subcore_barrier()

    grow0 = c * HALF + row0
    pltpu.async_copy(acc_sh.at[pl.ds(row0, RPT)],
                     xo_hbm.at[pl.ds(grow0, RPT)], zsem)
    pltpu.make_async_copy(acc_sh.at[pl.ds(row0, RPT)],
                          xo_hbm.at[pl.ds(grow0, RPT)], zsem).wait()

    @pl.when(s == NS - 1)
    def _wtail():
        r0 = RPT * NS
        g0 = c * HALF + r0
        pltpu.async_copy(acc_sh.at[pl.ds(r0, 8)],
                         xo_hbm.at[pl.ds(g0, 8)], zsem)
        pltpu.make_async_copy(acc_sh.at[pl.ds(r0, 8)],
                              xo_hbm.at[pl.ds(g0, 8)], zsem).wait()


def _gamma_body(x0_hbm, x1_hbm, x2_hbm, x3_hbm, u_hbm, i_hbm, out_hbm,
                uidx, iidx, u0, u1, u2, u3, i0, i1, i2, i3, obuf, gsem,
                n_pairs):
    c = lax.axis_index("c")
    s = lax.axis_index("s")
    wid = s * NC + c
    ppt = n_pairs // NW
    pbase = wid * ppt
    lane = lax.iota(_i32, 16)
    nusers = jnp.full((16,), N_USERS, _i32)
    xt = (x0_hbm, x1_hbm, x2_hbm, x3_hbm)
    ub = (u0, u1, u2, u3)
    ib = (i0, i1, i2, i3)

    for cc in range(ppt // GP):
        base = pbase + cc * GP
        pltpu.sync_copy(u_hbm.at[pl.ds(base, GP)], uidx)
        pltpu.sync_copy(i_hbm.at[pl.ds(base, GP)], iidx)

        def _adj(i, _):
            iidx[pl.ds(i * 16, 16)] = iidx[pl.ds(i * 16, 16)] + nusers
            return 0

        lax.fori_loop(0, GP // 16, _adj, 0)
        for t in range(4):
            pltpu.async_copy(xt[t].at[uidx], ub[t], gsem)
            pltpu.async_copy(xt[t].at[iidx], ib[t], gsem)
        for t in range(4):
            pltpu.make_async_copy(xt[t].at[uidx], ub[t], gsem).wait()
            pltpu.make_async_copy(xt[t].at[iidx], ib[t], gsem).wait()

        def _gb(g, _):
            out = jnp.zeros((16,), _f32)
            for l in range(16):
                e = g * 16 + l
                acc = jnp.zeros((16,), _f32)
                for j in range(NG):
                    us = (u0[e, pl.ds(j * 16, 16)] + u1[e, pl.ds(j * 16, 16)]
                          + u2[e, pl.ds(j * 16, 16)] + u3[e, pl.ds(j * 16, 16)])
                    vs = (i0[e, pl.ds(j * 16, 16)] + i1[e, pl.ds(j * 16, 16)]
                          + i2[e, pl.ds(j * 16, 16)] + i3[e, pl.ds(j * 16, 16)])
                    acc = acc + us * vs
                ssum = jnp.sum(acc)
                out = jnp.where(lane == l, jnp.full((16,), ssum), out)
            obuf[pl.ds(cc * GP + g * 16, 16)] = out * (1.0 / 64.0)
            return 0

        lax.fori_loop(0, GP // 16, _gb, 0)

    pltpu.sync_copy(obuf, out_hbm.at[pl.ds(pbase, ppt)])


def _make_layer():
    mesh = plsc.VectorSubcoreMesh(core_axis_name="c", subcore_axis_name="s")
    return pl.kernel(
        _layer_body,
        out_type=jax.ShapeDtypeStruct((NN, D), _f32),
        mesh=mesh,
        compiler_params=pltpu.CompilerParams(needs_layout_passes=False),
        scratch_types=[
            pltpu.VMEM((EBLK,), _i32),
            pltpu.VMEM((EBLK,), _i32),
            pltpu.VMEM((EBLK,), _f32),
            pltpu.VMEM((SEL_PAD,), _i32),
            pltpu.VMEM((SEL_PAD,), _i32),
            pltpu.VMEM((SEL_PAD,), _f32),
            pltpu.VMEM((G,), _i32),
            pltpu.VMEM((G,), _i32),
            pltpu.VMEM((G,), _i32),
            pltpu.VMEM((G,), _i32),
            pltpu.VMEM((G, D), _f32),
            pltpu.VMEM((G, D), _f32),
            pltpu.VMEM((ZC, D), _f32),
            pltpu.VMEM_SHARED((ACC_ROWS, D), _f32),
            pltpu.SemaphoreType.DMA,
            pltpu.SemaphoreType.DMA,
            pltpu.SemaphoreType.DMA,
            pltpu.SemaphoreType.DMA,
            pltpu.SemaphoreType.DMA,
        ],
    )


def _make_gamma(n_pairs):
    mesh = plsc.VectorSubcoreMesh(core_axis_name="c", subcore_axis_name="s")
    body = lambda *refs: _gamma_body(*refs, n_pairs=n_pairs)
    return pl.kernel(
        body,
        out_type=jax.ShapeDtypeStruct((n_pairs,), _f32),
        mesh=mesh,
        compiler_params=pltpu.CompilerParams(needs_layout_passes=False),
        scratch_types=(
            [pltpu.VMEM((GP,), _i32),
             pltpu.VMEM((GP,), _i32)]
            + [pltpu.VMEM((GP, D), _f32) for _ in range(8)]
            + [pltpu.VMEM((n_pairs // NW,), _f32),
               pltpu.SemaphoreType.DMA]
        ),
    )


def kernel(user_emb, item_emb, edge_weight, users, items, edge_index):
    x0 = jnp.concatenate([user_emb, item_emb], axis=0)
    dst = edge_index[0].astype(_i32)
    src = edge_index[1].astype(_i32)
    w = edge_weight
    layer = _make_layer()
    xs = [x0]
    for _ in range(N_LAYERS):
        xs.append(layer(xs[-1], dst, src, w))
    gamma = _make_gamma(users.shape[0])
    return gamma(xs[0], xs[1], xs[2], xs[3],
                 users.astype(_i32), items.astype(_i32))


# speedup vs baseline: 5.2800x; 1.0648x over previous
import jax
import jax.numpy as jnp
from jax import lax
from jax.experimental import pallas as pl
from jax.experimental.pallas import tpu as pltpu
from jax.experimental.pallas import tpu_sc as plsc

N_USERS = 5000
N_ITEMS = 5000
NN = N_USERS + N_ITEMS
NE = 320000
D = 128
NG = D // 16
N_LAYERS = 3
NC, NS, LANES = 2, 16, 16
NW = NC * NS
ESL = NE // NS
NPASS = 2
PESL = ESL // NPASS
HALF = NN // NC
G = 128
ACC_ROWS = HALF + 16
SEL_PAD = PESL + G + 16
RPT = 312
ZC = 24
EBLK = 2000
GP = 64

_f32 = jnp.float32
_i32 = jnp.int32


def _layer_body(x_hbm, dst_hbm, src_hbm, w_hbm, xo_hbm,
                dst_v, src_v, w_v, sdst, ssrc, sw,
                gidx0, gidx1, gidx2, sidx0, sidx1, sidx2,
                rows0, rows1, rows2,
                zbuf, acc_sh, gsem0, gsem1, gsem2, ssem0, ssem1, ssem2,
                zsem):
    c = lax.axis_index("c")
    s = lax.axis_index("s")
    ebase = s * ESL

    zero_f = jnp.zeros((16,), _f32)
    zero_i = jnp.zeros((16,), _i32)
    trash = jnp.full((16,), HALF, _i32)
    lane = lax.iota(_i32, 16)

    for r in range(ZC):
        for j in range(NG):
            zbuf[r, pl.ds(j * 16, 16)] = zero_f
    row0 = s * RPT

    def _zc(i, _):
        pltpu.async_copy(zbuf, acc_sh.at[pl.ds(row0 + i * ZC, ZC)], zsem)
        return 0

    lax.fori_loop(0, RPT // ZC, _zc, 0)

    @pl.when(s == NS - 1)
    def _ztail():
        pltpu.async_copy(zbuf.at[pl.ds(0, 8)],
                         acc_sh.at[pl.ds(RPT * NS, 8)], zsem)

    def _zw(i, _):
        pltpu.make_async_copy(zbuf, acc_sh.at[pl.ds(row0, ZC)], zsem).wait()
        return 0

    lax.fori_loop(0, RPT // ZC, _zw, 0)

    @pl.when(s == NS - 1)
    def _ztailw():
        pltpu.make_async_copy(zbuf.at[pl.ds(0, 8)],
                              acc_sh.at[pl.ds(RPT * NS, 8)], zsem).wait()

    plsc.subcore_barrier()

    halfbase = c * HALF
    dump = jnp.full((16,), SEL_PAD - 16, _i32) + lane

    gidx = (gidx0, gidx1, gidx2)
    sidx = (sidx0, sidx1, sidx2)
    rows = (rows0, rows1, rows2)
    gsem = (gsem0, gsem1, gsem2)
    ssem = (ssem0, ssem1, ssem2)

    def _prep(k, b):
        base = k * G

        def _ic(i, _):
            gidx[b][pl.ds(i * 16, 16)] = ssrc[pl.ds(base + i * 16, 16)]
            sidx[b][pl.ds(i * 16, 16)] = sdst[pl.ds(base + i * 16, 16)]
            return 0

        lax.fori_loop(0, G // 16, _ic, 0)
        pltpu.async_copy(x_hbm.at[gidx[b]], rows[b], gsem[b])

    def _scale(k, b):
        def _eb(i, _):
            wv = sw[pl.ds(k * G + i * 16, 16)]
            for l in range(16):
                wl = jnp.full((16,), wv[l])
                for j in range(NG):
                    rows[b][i * 16 + l, pl.ds(j * 16, 16)] = (
                        rows[b][i * 16 + l, pl.ds(j * 16, 16)] * wl)
            return 0

        lax.fori_loop(0, G // 16, _eb, 0)

    for p in range(NPASS):
        pbase = ebase + p * PESL

        def _blk(bi, cnt):
            pltpu.async_copy(dst_hbm.at[pl.ds(pbase + bi * EBLK, EBLK)],
                             dst_v, zsem)
            pltpu.async_copy(src_hbm.at[pl.ds(pbase + bi * EBLK, EBLK)],
                             src_v, zsem)
            pltpu.async_copy(w_hbm.at[pl.ds(pbase + bi * EBLK, EBLK)],
                             w_v, zsem)
            pltpu.make_async_copy(dst_hbm.at[pl.ds(pbase, EBLK)],
                                  dst_v, zsem).wait()
            pltpu.make_async_copy(src_hbm.at[pl.ds(pbase, EBLK)],
                                  src_v, zsem).wait()
            pltpu.make_async_copy(w_hbm.at[pl.ds(pbase, EBLK)],
                                  w_v, zsem).wait()

            def _fb(i, cnt):
                dv = dst_v[pl.ds(i * 16, 16)]
                loc = dv - halfbase
                m = (loc >= 0) & (loc < HALF)
                cum = plsc.cumsum(m.astype(_i32))
                pos = jnp.where(m, cnt + cum - 1, dump)
                plsc.store_scatter(sdst, [pos], loc)
                plsc.store_scatter(ssrc, [pos], src_v[pl.ds(i * 16, 16)])
                plsc.store_scatter(sw, [pos], w_v[pl.ds(i * 16, 16)])
                return cnt + cum[15]

            return lax.fori_loop(0, EBLK // 16, _fb, cnt)

        cnt = lax.fori_loop(0, PESL // EBLK, _blk, _i32(0))

        for i in range(G // 16):
            pidx = cnt + i * 16 + lane
            plsc.store_scatter(sdst, [pidx], trash)
            plsc.store_scatter(ssrc, [pidx], zero_i)
            plsc.store_scatter(sw, [pidx], zero_f)

        nch = jnp.maximum((cnt + G - 1) // G, 1)

        _prep(_i32(0), 0)

        @pl.when(_i32(1) < nch)
        def _p1():
            _prep(_i32(1), 1)

        def _round(r, _):
            for b in range(3):
                k = r * 3 + b

                @pl.when(k < nch)
                def _():
                    pltpu.make_async_copy(
                        x_hbm.at[gidx[b]], rows[b], gsem[b]).wait()

                    @pl.when(k > 1)
                    def _():
                        bb = (b + 1) % 3
                        pltpu.make_async_copy(
                            rows[bb], acc_sh.at[sidx[bb]], ssem[bb]).wait()

                    @pl.when(k + 2 < nch)
                    def _():
                        _prep(k + 2, (b + 2) % 3)

                    _scale(k, b)
                    pltpu.async_copy(rows[b], acc_sh.at[sidx[b]],
                                     ssem[b], add=True)
            return 0

        lax.fori_loop(0, (nch + 2) // 3, _round, 0)

        for d in range(2):
            kd = nch - 2 + d

            @pl.when(kd >= 0)
            def _dd():
                for b in range(3):
                    @pl.when(kd % 3 == b)
                    def _db():
                        pltpu.make_async_copy(
                            rows[b], acc_sh.at[sidx[b]], ssem[b]).wait()

    plsc.subcore_barrier()

    grow0 = c * HALF + row0
    pltpu.async_copy(acc_sh.at[pl.ds(row0, RPT)],
                     xo_hbm.at[pl.ds(grow0, RPT)], zsem)
    pltpu.make_async_copy(acc_sh.at[pl.ds(row0, RPT)],
                          xo_hbm.at[pl.ds(grow0, RPT)], zsem).wait()

    @pl.when(s == NS - 1)
    def _wtail():
        r0 = RPT * NS
        g0 = c * HALF + r0
        pltpu.async_copy(acc_sh.at[pl.ds(r0, 8)],
                         xo_hbm.at[pl.ds(g0, 8)], zsem)
        pltpu.make_async_copy(acc_sh.at[pl.ds(r0, 8)],
                              xo_hbm.at[pl.ds(g0, 8)], zsem).wait()


def _gamma_body(x0_hbm, x1_hbm, x2_hbm, x3_hbm, u_hbm, i_hbm, out_hbm,
                uidx, iidx, u0, u1, u2, u3, i0, i1, i2, i3, obuf, gsem,
                n_pairs):
    c = lax.axis_index("c")
    s = lax.axis_index("s")
    wid = s * NC + c
    ppt = n_pairs // NW
    pbase = wid * ppt
    lane = lax.iota(_i32, 16)
    nusers = jnp.full((16,), N_USERS, _i32)
    xt = (x0_hbm, x1_hbm, x2_hbm, x3_hbm)
    ub = (u0, u1, u2, u3)
    ib = (i0, i1, i2, i3)

    for cc in range(ppt // GP):
        base = pbase + cc * GP
        pltpu.sync_copy(u_hbm.at[pl.ds(base, GP)], uidx)
        pltpu.sync_copy(i_hbm.at[pl.ds(base, GP)], iidx)

        def _adj(i, _):
            iidx[pl.ds(i * 16, 16)] = iidx[pl.ds(i * 16, 16)] + nusers
            return 0

        lax.fori_loop(0, GP // 16, _adj, 0)
        for t in range(4):
            pltpu.async_copy(xt[t].at[uidx], ub[t], gsem)
            pltpu.async_copy(xt[t].at[iidx], ib[t], gsem)
        for t in range(4):
            pltpu.make_async_copy(xt[t].at[uidx], ub[t], gsem).wait()
            pltpu.make_async_copy(xt[t].at[iidx], ib[t], gsem).wait()

        def _gb(g, _):
            out = jnp.zeros((16,), _f32)
            for l in range(16):
                e = g * 16 + l
                acc = jnp.zeros((16,), _f32)
                for j in range(NG):
                    us = (u0[e, pl.ds(j * 16, 16)] + u1[e, pl.ds(j * 16, 16)]
                          + u2[e, pl.ds(j * 16, 16)] + u3[e, pl.ds(j * 16, 16)])
                    vs = (i0[e, pl.ds(j * 16, 16)] + i1[e, pl.ds(j * 16, 16)]
                          + i2[e, pl.ds(j * 16, 16)] + i3[e, pl.ds(j * 16, 16)])
                    acc = acc + us * vs
                ssum = jnp.sum(acc)
                out = jnp.where(lane == l, jnp.full((16,), ssum), out)
            obuf[pl.ds(cc * GP + g * 16, 16)] = out * (1.0 / 64.0)
            return 0

        lax.fori_loop(0, GP // 16, _gb, 0)

    pltpu.sync_copy(obuf, out_hbm.at[pl.ds(pbase, ppt)])


def _make_layer():
    mesh = plsc.VectorSubcoreMesh(core_axis_name="c", subcore_axis_name="s")
    return pl.kernel(
        _layer_body,
        out_type=jax.ShapeDtypeStruct((NN, D), _f32),
        mesh=mesh,
        compiler_params=pltpu.CompilerParams(needs_layout_passes=False),
        scratch_types=[
            pltpu.VMEM((EBLK,), _i32),
            pltpu.VMEM((EBLK,), _i32),
            pltpu.VMEM((EBLK,), _f32),
            pltpu.VMEM((SEL_PAD,), _i32),
            pltpu.VMEM((SEL_PAD,), _i32),
            pltpu.VMEM((SEL_PAD,), _f32),
            pltpu.VMEM((G,), _i32),
            pltpu.VMEM((G,), _i32),
            pltpu.VMEM((G,), _i32),
            pltpu.VMEM((G,), _i32),
            pltpu.VMEM((G,), _i32),
            pltpu.VMEM((G,), _i32),
            pltpu.VMEM((G, D), _f32),
            pltpu.VMEM((G, D), _f32),
            pltpu.VMEM((G, D), _f32),
            pltpu.VMEM((ZC, D), _f32),
            pltpu.VMEM_SHARED((ACC_ROWS, D), _f32),
            pltpu.SemaphoreType.DMA,
            pltpu.SemaphoreType.DMA,
            pltpu.SemaphoreType.DMA,
            pltpu.SemaphoreType.DMA,
            pltpu.SemaphoreType.DMA,
            pltpu.SemaphoreType.DMA,
            pltpu.SemaphoreType.DMA,
        ],
    )


def _make_gamma(n_pairs):
    mesh = plsc.VectorSubcoreMesh(core_axis_name="c", subcore_axis_name="s")
    body = lambda *refs: _gamma_body(*refs, n_pairs=n_pairs)
    return pl.kernel(
        body,
        out_type=jax.ShapeDtypeStruct((n_pairs,), _f32),
        mesh=mesh,
        compiler_params=pltpu.CompilerParams(needs_layout_passes=False),
        scratch_types=(
            [pltpu.VMEM((GP,), _i32),
             pltpu.VMEM((GP,), _i32)]
            + [pltpu.VMEM((GP, D), _f32) for _ in range(8)]
            + [pltpu.VMEM((n_pairs // NW,), _f32),
               pltpu.SemaphoreType.DMA]
        ),
    )


def kernel(user_emb, item_emb, edge_weight, users, items, edge_index):
    x0 = jnp.concatenate([user_emb, item_emb], axis=0)
    dst = edge_index[0].astype(_i32)
    src = edge_index[1].astype(_i32)
    w = edge_weight
    layer = _make_layer()
    xs = [x0]
    for _ in range(N_LAYERS):
        xs.append(layer(xs[-1], dst, src, w))
    gamma = _make_gamma(users.shape[0])
    return gamma(xs[0], xs[1], xs[2], xs[3],
                 users.astype(_i32), items.astype(_i32))
